# baseline (speedup 1.0000x reference)
import functools

import jax
import jax.numpy as jnp
from jax import lax
from jax.experimental import pallas as pl
from jax.experimental.pallas import tpu as pltpu
from jax.experimental.pallas import tpu_sc as plsc

_B_BLK = 128


def _tc_broadcast_kernel(table_ref, out_ref):
    row = table_ref[0, :]
    out_ref[...] = jnp.broadcast_to(row[None, None, :], out_ref.shape)


def _tc_broadcast(table, n_rows, L, C):
    grid = (n_rows // (_B_BLK * L),)
    return pl.pallas_call(
        _tc_broadcast_kernel,
        grid=grid,
        in_specs=[pl.BlockSpec((1, C), lambda i: (0, 0))],
        out_specs=pl.BlockSpec((_B_BLK, L, C), lambda i: (i, 0, 0)),
        out_shape=jax.ShapeDtypeStruct((n_rows // L, L, C), table.dtype),
    )(table)


def _sc_broadcast(table, n_rows, C):
    NC, NS = 2, 16
    NW = NC * NS
    rows_per_w = n_rows // NW
    R = 800
    n_dma = rows_per_w // R
    assert n_dma * R == rows_per_w and rows_per_w * NW == n_rows
    mesh = plsc.VectorSubcoreMesh(core_axis_name="c", subcore_axis_name="s")

    @functools.partial(
        pl.kernel,
        mesh=mesh,
        out_type=jax.ShapeDtypeStruct((n_rows, C), jnp.float32),
        scratch_types=[
            pltpu.VMEM((1, C), jnp.float32),
            pltpu.VMEM((R, C), jnp.float32),
            pltpu.SemaphoreType.DMA,
        ],
    )
    def k(table_hbm, out_hbm, row_v, chunk_v, sem):
        wid = lax.axis_index("s") * NC + lax.axis_index("c")
        pltpu.sync_copy(table_hbm, row_v)
        vecs = [row_v[0, pl.ds(j * 16, 16)] for j in range(C // 16)]

        def fill(r, carry):
            for j in range(C // 16):
                chunk_v[r, pl.ds(j * 16, 16)] = vecs[j]
            return carry

        lax.fori_loop(0, R, fill, 0)

        base = wid * rows_per_w
        copies = [
            pltpu.make_async_copy(chunk_v, out_hbm.at[pl.ds(base + i * R, R)], sem)
            for i in range(n_dma)
        ]
        for cp in copies:
            cp.start()
        for cp in copies:
            cp.wait()

    return k(table)


_RB_SMALL = 400
_RB_BIG = 12800


def _tc_dma_broadcast(table, n_rows, C):
    n_small = _RB_BIG // _RB_SMALL
    n_big = n_rows // _RB_BIG - 1

    def body(table_ref, out_hbm, small, big, sem):
        row = table_ref[0, :][None, :]
        small[...] = jnp.broadcast_to(row, small.shape)
        copies = [
            pltpu.make_async_copy(
                small, out_hbm.at[pl.ds(i * _RB_SMALL, _RB_SMALL)], sem
            )
            for i in range(n_small)
        ]
        for cp in copies:
            cp.start()
        big[...] = jnp.broadcast_to(row, big.shape)
        big_copies = [
            pltpu.make_async_copy(
                big, out_hbm.at[pl.ds(_RB_BIG + i * _RB_BIG, _RB_BIG)], sem
            )
            for i in range(n_big)
        ]
        for cp in big_copies:
            cp.start()
        for cp in copies + big_copies:
            cp.wait()

    return pl.pallas_call(
        body,
        in_specs=[pl.BlockSpec((1, C), lambda: (0, 0))],
        out_specs=pl.BlockSpec(memory_space=pl.ANY),
        out_shape=jax.ShapeDtypeStruct((n_rows, C), table.dtype),
        scratch_shapes=[
            pltpu.VMEM((_RB_SMALL, C), jnp.float32),
            pltpu.VMEM((_RB_BIG, C), jnp.float32),
            pltpu.SemaphoreType.DMA,
        ],
    )(table)


def kernel(y, table):
    B, L, C = y.shape[0], y.shape[-2], y.shape[-1]
    return _sc_broadcast(table, B * L, C).reshape(B, L, C)


# speedup vs baseline: 1.2646x; 1.2646x over previous
import jax
import jax.numpy as jnp
from jax.experimental import pallas as pl

_B_BLK = 64


def _broadcast_kernel(table_ref, out_ref):
    row = table_ref[0, :]
    out_ref[...] = jnp.broadcast_to(row[None, None, :], out_ref.shape)


def kernel(y, table):
    B, L, C = y.shape[0], y.shape[-2], y.shape[-1]
    return pl.pallas_call(
        _broadcast_kernel,
        grid=(B // _B_BLK,),
        in_specs=[pl.BlockSpec((1, C), lambda i: (0, 0))],
        out_specs=pl.BlockSpec((_B_BLK, L, C), lambda i: (i, 0, 0)),
        out_shape=jax.ShapeDtypeStruct((B, L, C), table.dtype),
    )(table)
